# Initial kernel scaffold; baseline (speedup 1.0000x reference)
import jax
import jax.numpy as jnp
from jax.experimental import pallas as pl


def kernel(x, edge_index, gcn_W, gcn_b, Wq, bq, Wk, bk, Wv, bv, Ws, bs):
    raise NotImplementedError("write your pallas kernel here")


# speedup vs baseline: 3.6212x; 3.6212x over previous
import functools
import math

import jax
import jax.numpy as jnp
from jax import lax
from jax.experimental import pallas as pl
from jax.experimental.pallas import tpu as pltpu
from jax.experimental.pallas import tpu_sc as plsc

N = 10000
D = 128
E = 320000

NC = 2
NS = 16
NW = NC * NS
EPW = E // NW
CD = 80
NCHUNK = EPW // CD
NPAD = 10240
RPT = NPAD // NS

INV_SQRT_D = 1.0 / math.sqrt(D)
NEG_BIG = -3.0e38

_MESH = plsc.VectorSubcoreMesh(core_axis_name="c", subcore_axis_name="s")


@functools.partial(
    pl.kernel,
    compiler_params=pltpu.CompilerParams(needs_layout_passes=False),
    out_type=jax.ShapeDtypeStruct((NC, NPAD), jnp.float32),
    mesh=_MESH,
    scratch_types=[
        pltpu.VMEM((CD,), jnp.int32),
        pltpu.VMEM((CD,), jnp.float32),
        pltpu.VMEM_SHARED((NPAD,), jnp.float32),
    ],
)
def _sc_degree(dst_hbm, z1_hbm, out_hbm, didx_v, ones_v, deg_sp):
    c = lax.axis_index("c")
    s = lax.axis_index("s")
    wid = c * NS + s
    pltpu.sync_copy(z1_hbm.at[pl.ds(s * RPT, RPT)], deg_sp.at[pl.ds(s * RPT, RPT)])
    for g in range(CD // 16):
        ones_v[pl.ds(g * 16, 16)] = jnp.full((16,), 1.0, jnp.float32)
    plsc.subcore_barrier()
    base = wid * EPW

    def chunk(i, carry):
        off = base + i * CD
        pltpu.sync_copy(dst_hbm.at[pl.ds(off, CD)], didx_v)
        pltpu.sync_copy(ones_v, deg_sp.at[didx_v], add=True)
        return carry

    lax.fori_loop(0, NCHUNK, chunk, 0)
    plsc.subcore_barrier()
    pltpu.sync_copy(deg_sp.at[pl.ds(s * RPT, RPT)], out_hbm.at[c, pl.ds(s * RPT, RPT)])


@functools.partial(
    pl.kernel,
    compiler_params=pltpu.CompilerParams(needs_layout_passes=False),
    out_type=jax.ShapeDtypeStruct((NC, NPAD, D), jnp.float32),
    mesh=_MESH,
    scratch_types=[
        pltpu.VMEM((CD,), jnp.int32),
        pltpu.VMEM((CD,), jnp.int32),
        pltpu.VMEM((CD, D), jnp.float32),
        pltpu.VMEM_SHARED((NPAD, D), jnp.float32),
        pltpu.SemaphoreType.DMA,
    ],
)
def _sc_rowagg(src_hbm, dst_hbm, hs_hbm, z2_hbm, out_hbm,
               sidx_v, didx_v, rows_v, agg_sp, sem):
    c = lax.axis_index("c")
    s = lax.axis_index("s")
    wid = c * NS + s
    pltpu.sync_copy(z2_hbm.at[pl.ds(s * RPT, RPT)], agg_sp.at[pl.ds(s * RPT, RPT)])
    plsc.subcore_barrier()
    base = wid * EPW

    def chunk(i, carry):
        off = base + i * CD
        pltpu.sync_copy(src_hbm.at[pl.ds(off, CD)], sidx_v)
        pltpu.sync_copy(dst_hbm.at[pl.ds(off, CD)], didx_v)
        pltpu.async_copy(hs_hbm.at[sidx_v], rows_v, sem).wait()
        pltpu.sync_copy(rows_v, agg_sp.at[didx_v], add=True)
        return carry

    lax.fori_loop(0, NCHUNK, chunk, 0)
    plsc.subcore_barrier()
    pltpu.sync_copy(agg_sp.at[pl.ds(s * RPT, RPT)], out_hbm.at[c, pl.ds(s * RPT, RPT)])


@functools.partial(
    pl.kernel,
    compiler_params=pltpu.CompilerParams(needs_layout_passes=False),
    out_type=(
        jax.ShapeDtypeStruct((E,), jnp.float32),
        jax.ShapeDtypeStruct((NW * 16,), jnp.float32),
    ),
    mesh=_MESH,
    scratch_types=[
        pltpu.VMEM((CD,), jnp.int32),
        pltpu.VMEM((CD,), jnp.int32),
        pltpu.VMEM((CD, D), jnp.float32),
        pltpu.VMEM((CD, D), jnp.float32),
        pltpu.VMEM((CD,), jnp.float32),
        pltpu.VMEM((16,), jnp.float32),
        pltpu.SemaphoreType.DMA,
    ],
)
def _sc_alpha(src_hbm, dst_hbm, q_hbm, k_hbm, alpha_hbm, maxes_hbm,
              sidx_v, didx_v, qrows_v, krows_v, alpha_v, mv_v, sem):
    c = lax.axis_index("c")
    s = lax.axis_index("s")
    wid = c * NS + s
    base = wid * EPW

    def chunk(i, m):
        off = base + i * CD
        pltpu.sync_copy(src_hbm.at[pl.ds(off, CD)], sidx_v)
        pltpu.sync_copy(dst_hbm.at[pl.ds(off, CD)], didx_v)
        cp_q = pltpu.async_copy(q_hbm.at[didx_v], qrows_v, sem)
        cp_k = pltpu.async_copy(k_hbm.at[sidx_v], krows_v, sem)
        cp_q.wait()
        cp_k.wait()
        for g in range(CD // 16):
            rows16 = lax.iota(jnp.int32, 16) + g * 16
            acc = jnp.zeros((16,), jnp.float32)
            for j in range(D):
                colj = jnp.full((16,), j, jnp.int32)
                qc = plsc.load_gather(qrows_v, [rows16, colj])
                kc = plsc.load_gather(krows_v, [rows16, colj])
                acc = acc + qc * kc
            av = acc * INV_SQRT_D
            alpha_v[pl.ds(g * 16, 16)] = av
            m = jnp.maximum(m, av)
        pltpu.sync_copy(alpha_v, alpha_hbm.at[pl.ds(off, CD)])
        return m

    m = lax.fori_loop(0, NCHUNK, chunk, jnp.full((16,), NEG_BIG, jnp.float32))
    mv_v[...] = m
    pltpu.sync_copy(mv_v, maxes_hbm.at[pl.ds(wid * 16, 16)])


@functools.partial(
    pl.kernel,
    compiler_params=pltpu.CompilerParams(needs_layout_passes=False),
    out_type=(
        jax.ShapeDtypeStruct((NC, NPAD, D), jnp.float32),
        jax.ShapeDtypeStruct((NC, NPAD), jnp.float32),
    ),
    mesh=_MESH,
    scratch_types=[
        pltpu.VMEM((CD,), jnp.int32),
        pltpu.VMEM((CD,), jnp.int32),
        pltpu.VMEM((CD, D), jnp.float32),
        pltpu.VMEM((CD,), jnp.float32),
        pltpu.VMEM((NW * 16,), jnp.float32),
        pltpu.VMEM_SHARED((NPAD, D), jnp.float32),
        pltpu.VMEM_SHARED((NPAD,), jnp.float32),
        pltpu.SemaphoreType.DMA,
    ],
)
def _sc_wagg(src_hbm, dst_hbm, v_hbm, alpha_hbm, maxes_hbm, z2_hbm, z1_hbm,
             wout_hbm, dout_hbm,
             sidx_v, didx_v, rows_v, av_v, maxv_v, wsp, dsp, sem):
    c = lax.axis_index("c")
    s = lax.axis_index("s")
    wid = c * NS + s
    pltpu.sync_copy(z2_hbm.at[pl.ds(s * RPT, RPT)], wsp.at[pl.ds(s * RPT, RPT)])
    pltpu.sync_copy(z1_hbm.at[pl.ds(s * RPT, RPT)], dsp.at[pl.ds(s * RPT, RPT)])
    pltpu.sync_copy(maxes_hbm, maxv_v)
    m = jnp.full((16,), NEG_BIG, jnp.float32)
    for w in range(NW):
        m = jnp.maximum(m, maxv_v[pl.ds(w * 16, 16)])
    gmax = jnp.max(m)
    plsc.subcore_barrier()
    base = wid * EPW

    def chunk(i, carry):
        off = base + i * CD
        pltpu.sync_copy(src_hbm.at[pl.ds(off, CD)], sidx_v)
        pltpu.sync_copy(dst_hbm.at[pl.ds(off, CD)], didx_v)
        pltpu.sync_copy(alpha_hbm.at[pl.ds(off, CD)], av_v)
        pltpu.async_copy(v_hbm.at[sidx_v], rows_v, sem).wait()
        for g in range(CD // 16):
            a16 = jnp.exp(av_v[pl.ds(g * 16, 16)] - gmax)
            av_v[pl.ds(g * 16, 16)] = a16
            rows16 = lax.iota(jnp.int32, 16) + g * 16
            for j in range(D):
                colj = jnp.full((16,), j, jnp.int32)
                col = plsc.load_gather(rows_v, [rows16, colj])
                plsc.store_scatter(rows_v, [rows16, colj], col * a16)
        pltpu.sync_copy(rows_v, wsp.at[didx_v], add=True)
        pltpu.sync_copy(av_v, dsp.at[didx_v], add=True)
        return carry

    lax.fori_loop(0, NCHUNK, chunk, 0)
    plsc.subcore_barrier()
    pltpu.sync_copy(wsp.at[pl.ds(s * RPT, RPT)], wout_hbm.at[c, pl.ds(s * RPT, RPT)])
    pltpu.sync_copy(dsp.at[pl.ds(s * RPT, RPT)], dout_hbm.at[c, pl.ds(s * RPT, RPT)])


BLK = 1000
GRID = N // BLK


def _tc_mm_body(x_ref, w_ref, o_ref):
    o_ref[...] = jnp.dot(x_ref[...], w_ref[...], preferred_element_type=jnp.float32)


def _tc_matmul(x, w):
    return pl.pallas_call(
        _tc_mm_body,
        grid=(GRID,),
        in_specs=[
            pl.BlockSpec((BLK, D), lambda i: (i, 0)),
            pl.BlockSpec((D, D), lambda i: (0, 0)),
        ],
        out_specs=pl.BlockSpec((BLK, D), lambda i: (i, 0)),
        out_shape=jax.ShapeDtypeStruct((N, D), jnp.float32),
    )(x, w)


def _tc_scale_body(h0_ref, degp_ref, hs_ref):
    deg = degp_ref[0] + degp_ref[1] + 1.0
    dinv = lax.rsqrt(deg)
    hs_ref[...] = h0_ref[...] * dinv


def _tc_scale(h0, degp):
    return pl.pallas_call(
        _tc_scale_body,
        grid=(GRID,),
        in_specs=[
            pl.BlockSpec((BLK, D), lambda i: (i, 0)),
            pl.BlockSpec((NC, BLK, 1), lambda i: (0, i, 0)),
        ],
        out_specs=pl.BlockSpec((BLK, D), lambda i: (i, 0)),
        out_shape=jax.ShapeDtypeStruct((N, D), jnp.float32),
    )(h0, degp)


def _tc_qkvs_body(aggp_ref, h0_ref, degp_ref, gb_ref,
                  wq_ref, bq_ref, wk_ref, bk_ref, wv_ref, bv_ref, ws_ref, bs_ref,
                  q_ref, k_ref, v_ref, s_ref):
    deg = degp_ref[0] + degp_ref[1] + 1.0
    dinv = lax.rsqrt(deg)
    agg = aggp_ref[0] + aggp_ref[1]
    h = dinv * agg + (dinv * dinv) * h0_ref[...] + gb_ref[...]
    h = jnp.where(h >= 0, h, 0.01 * h)
    q_ref[...] = jnp.dot(h, wq_ref[...], preferred_element_type=jnp.float32) + bq_ref[...]
    k_ref[...] = jnp.dot(h, wk_ref[...], preferred_element_type=jnp.float32) + bk_ref[...]
    v_ref[...] = jnp.dot(h, wv_ref[...], preferred_element_type=jnp.float32) + bv_ref[...]
    s_ref[...] = jnp.dot(h, ws_ref[...], preferred_element_type=jnp.float32) + bs_ref[...]


def _tc_qkvs(aggp, h0, degp, gb, wq, bq, wk, bk, wv, bv, ws, bs):
    wspec = pl.BlockSpec((D, D), lambda i: (0, 0))
    bspec = pl.BlockSpec((1, D), lambda i: (0, 0))
    nspec = pl.BlockSpec((BLK, D), lambda i: (i, 0))
    osh = jax.ShapeDtypeStruct((N, D), jnp.float32)
    return pl.pallas_call(
        _tc_qkvs_body,
        grid=(GRID,),
        in_specs=[
            pl.BlockSpec((NC, BLK, D), lambda i: (0, i, 0)),
            nspec,
            pl.BlockSpec((NC, BLK, 1), lambda i: (0, i, 0)),
            bspec, wspec, bspec, wspec, bspec, wspec, bspec, wspec, bspec,
        ],
        out_specs=[nspec, nspec, nspec, nspec],
        out_shape=[osh, osh, osh, osh],
    )(aggp, h0, degp, gb, wq, bq, wk, bk, wv, bv, ws, bs)


def _tc_final_body(waggp_ref, denomp_ref, s_ref, o_ref):
    denom = denomp_ref[0] + denomp_ref[1] + 1e-16
    wagg = waggp_ref[0] + waggp_ref[1]
    o_ref[...] = wagg / denom + s_ref[...]


def _tc_final(waggp, denomp, sarr):
    return pl.pallas_call(
        _tc_final_body,
        grid=(GRID,),
        in_specs=[
            pl.BlockSpec((NC, BLK, D), lambda i: (0, i, 0)),
            pl.BlockSpec((NC, BLK, 1), lambda i: (0, i, 0)),
            pl.BlockSpec((BLK, D), lambda i: (i, 0)),
        ],
        out_specs=pl.BlockSpec((BLK, D), lambda i: (i, 0)),
        out_shape=jax.ShapeDtypeStruct((N, D), jnp.float32),
    )(waggp, denomp, sarr)


def kernel(x, edge_index, gcn_W, gcn_b, Wq, bq, Wk, bk, Wv, bv, Ws, bs):
    src = edge_index[0]
    dst = edge_index[1]
    z1 = jnp.zeros((NPAD,), jnp.float32)
    z2 = jnp.zeros((NPAD, D), jnp.float32)

    degp = _sc_degree(dst, z1)
    h0 = _tc_matmul(x, gcn_W)
    degp_n = degp[:, :N].reshape(NC, N, 1)
    hs = _tc_scale(h0, degp_n)

    aggp = _sc_rowagg(src, dst, hs, z2)
    aggp_n = aggp[:, :N].reshape(NC, N, D)
    gb = gcn_b.reshape(1, D)
    q, k, v, sarr = _tc_qkvs(aggp_n, h0, degp_n, gb,
                             Wq, bq.reshape(1, D), Wk, bk.reshape(1, D),
                             Wv, bv.reshape(1, D), Ws, bs.reshape(1, D))

    alpha, maxes = _sc_alpha(src, dst, q, k)
    waggp, denomp = _sc_wagg(src, dst, v, alpha, maxes, z2, z1)
    out = _tc_final(waggp[:, :N].reshape(NC, N, D),
                    denomp[:, :N].reshape(NC, N, 1), sarr)
    return out


# speedup vs baseline: 17.8974x; 4.9424x over previous
import functools
import math

import jax
import jax.numpy as jnp
from jax import lax
from jax.experimental import pallas as pl
from jax.experimental.pallas import tpu as pltpu
from jax.experimental.pallas import tpu_sc as plsc

N = 10000
D = 128
E = 320000

NC = 2
NS = 16
NW = NC * NS
EPW = E // NW
CD = 80
NCHUNK = EPW // CD
NPAD = 10240
RPT = NPAD // NS

INV_SQRT_D = 1.0 / math.sqrt(D)
NEG_BIG = -3.0e38

_MESH = plsc.VectorSubcoreMesh(core_axis_name="c", subcore_axis_name="s")
_PARAMS = pltpu.CompilerParams(needs_layout_passes=False)


@functools.partial(
    pl.kernel,
    compiler_params=_PARAMS,
    out_type=jax.ShapeDtypeStruct((NC, NPAD), jnp.float32),
    mesh=_MESH,
    scratch_types=[
        pltpu.VMEM((NCHUNK, CD), jnp.int32),
        pltpu.VMEM((CD,), jnp.float32),
        pltpu.VMEM_SHARED((NPAD,), jnp.float32),
    ],
)
def _sc_degree(dst_hbm, z1_hbm, out_hbm, didx2_v, ones_v, deg_sp):
    c = lax.axis_index("c")
    s = lax.axis_index("s")
    wid = c * NS + s
    pltpu.sync_copy(z1_hbm.at[pl.ds(s * RPT, RPT)], deg_sp.at[pl.ds(s * RPT, RPT)])
    pltpu.sync_copy(dst_hbm.at[wid], didx2_v)
    for g in range(CD // 16):
        ones_v[pl.ds(g * 16, 16)] = jnp.full((16,), 1.0, jnp.float32)
    plsc.subcore_barrier()

    def chunk(i, carry):
        pltpu.sync_copy(ones_v, deg_sp.at[didx2_v.at[i]], add=True)
        return carry

    lax.fori_loop(0, NCHUNK, chunk, 0)
    plsc.subcore_barrier()
    pltpu.sync_copy(deg_sp.at[pl.ds(s * RPT, RPT)], out_hbm.at[c, pl.ds(s * RPT, RPT)])


@functools.partial(
    pl.kernel,
    compiler_params=_PARAMS,
    out_type=jax.ShapeDtypeStruct((NC, NPAD, D), jnp.float32),
    mesh=_MESH,
    scratch_types=[
        pltpu.VMEM((EPW,), jnp.int32),
        pltpu.VMEM((NCHUNK, CD), jnp.int32),
        pltpu.VMEM((2 * CD, D), jnp.float32),
        pltpu.VMEM_SHARED((NPAD, D), jnp.float32),
        pltpu.SemaphoreType.DMA,
        pltpu.SemaphoreType.DMA,
    ],
)
def _sc_rowagg(src_hbm, dst_hbm, hs_hbm, z2_hbm, out_hbm,
               sidx_v, didx2_v, rows_v, agg_sp, sem_a, sem_b):
    c = lax.axis_index("c")
    s = lax.axis_index("s")
    wid = c * NS + s
    pltpu.sync_copy(z2_hbm.at[pl.ds(s * RPT, RPT)], agg_sp.at[pl.ds(s * RPT, RPT)])
    pltpu.sync_copy(src_hbm.at[pl.ds(wid * EPW, EPW)], sidx_v)
    pltpu.sync_copy(dst_hbm.at[wid], didx2_v)
    plsc.subcore_barrier()

    def fire(i, base, sem):
        pltpu.async_copy(hs_hbm.at[sidx_v.at[pl.ds(i * CD, CD)]],
                         rows_v.at[pl.ds(base, CD)], sem)

    def drain(base, sem):
        pltpu.make_async_copy(hs_hbm.at[sidx_v.at[pl.ds(0, CD)]],
                              rows_v.at[pl.ds(base, CD)], sem).wait()

    def flush(i, base):
        pltpu.sync_copy(rows_v.at[pl.ds(base, CD)], agg_sp.at[didx2_v.at[i]], add=True)

    fire(0, 0, sem_a)

    def pair(p, carry):
        fire(2 * p + 1, CD, sem_b)
        drain(0, sem_a)
        flush(2 * p, 0)
        fire(2 * p + 2, 0, sem_a)
        drain(CD, sem_b)
        flush(2 * p + 1, CD)
        return carry

    lax.fori_loop(0, (NCHUNK - 1) // 2, pair, 0)
    drain(0, sem_a)
    flush(NCHUNK - 1, 0)
    plsc.subcore_barrier()
    pltpu.sync_copy(agg_sp.at[pl.ds(s * RPT, RPT)], out_hbm.at[c, pl.ds(s * RPT, RPT)])


@functools.partial(
    pl.kernel,
    compiler_params=_PARAMS,
    out_type=(
        jax.ShapeDtypeStruct((E,), jnp.float32),
        jax.ShapeDtypeStruct((NW * 16,), jnp.float32),
    ),
    mesh=_MESH,
    scratch_types=[
        pltpu.VMEM((EPW,), jnp.int32),
        pltpu.VMEM((EPW,), jnp.int32),
        pltpu.VMEM((2 * CD, D), jnp.float32),
        pltpu.VMEM((2 * CD, D), jnp.float32),
        pltpu.VMEM((EPW,), jnp.float32),
        pltpu.VMEM((16,), jnp.float32),
        pltpu.SemaphoreType.DMA,
        pltpu.SemaphoreType.DMA,
    ],
)
def _sc_alpha(src_hbm, dst_hbm, q_hbm, k_hbm, alpha_hbm, maxes_hbm,
              sidx_v, didx_v, q_v, k_v, av_v, mv_v, sem_a, sem_b):
    c = lax.axis_index("c")
    s = lax.axis_index("s")
    wid = c * NS + s
    pltpu.sync_copy(src_hbm.at[pl.ds(wid * EPW, EPW)], sidx_v)
    pltpu.sync_copy(dst_hbm.at[pl.ds(wid * EPW, EPW)], didx_v)
    lanes = lax.iota(jnp.int32, 16)

    def fire(i, base, sem):
        pltpu.async_copy(q_hbm.at[didx_v.at[pl.ds(i * CD, CD)]],
                         q_v.at[pl.ds(base, CD)], sem)
        pltpu.async_copy(k_hbm.at[sidx_v.at[pl.ds(i * CD, CD)]],
                         k_v.at[pl.ds(base, CD)], sem)

    def drain(base, sem):
        pltpu.make_async_copy(q_hbm.at[didx_v.at[pl.ds(0, CD)]],
                              q_v.at[pl.ds(base, CD)], sem).wait()
        pltpu.make_async_copy(k_hbm.at[sidx_v.at[pl.ds(0, CD)]],
                              k_v.at[pl.ds(base, CD)], sem).wait()

    def compute(i, base, m):
        def grp(g, m2):
            parts = []
            for el in range(16):
                e = base + g * 16 + el
                acc = q_v[e, pl.ds(0, 16)] * k_v[e, pl.ds(0, 16)]
                for j in range(1, D // 16):
                    acc = acc + q_v[e, pl.ds(j * 16, 16)] * k_v[e, pl.ds(j * 16, 16)]
                dot = jnp.sum(acc)
                parts.append(jnp.where(lanes == el, dot, 0.0))
            while len(parts) > 1:
                parts = [parts[2 * t] + parts[2 * t + 1]
                         for t in range(len(parts) // 2)]
            av = parts[0] * INV_SQRT_D
            av_v[pl.ds(i * CD + g * 16, 16)] = av
            return jnp.maximum(m2, av)

        return lax.fori_loop(0, CD // 16, grp, m)

    fire(0, 0, sem_a)

    def pair(p, m):
        fire(2 * p + 1, CD, sem_b)
        drain(0, sem_a)
        m = compute(2 * p, 0, m)
        fire(2 * p + 2, 0, sem_a)
        drain(CD, sem_b)
        m = compute(2 * p + 1, CD, m)
        return m

    m = lax.fori_loop(0, (NCHUNK - 1) // 2, pair,
                      jnp.full((16,), NEG_BIG, jnp.float32))
    drain(0, sem_a)
    m = compute(NCHUNK - 1, 0, m)
    pltpu.sync_copy(av_v, alpha_hbm.at[pl.ds(wid * EPW, EPW)])
    mv_v[...] = m
    pltpu.sync_copy(mv_v, maxes_hbm.at[pl.ds(wid * 16, 16)])


@functools.partial(
    pl.kernel,
    compiler_params=_PARAMS,
    out_type=(
        jax.ShapeDtypeStruct((NC, NPAD, D), jnp.float32),
        jax.ShapeDtypeStruct((NC, NPAD), jnp.float32),
    ),
    mesh=_MESH,
    scratch_types=[
        pltpu.VMEM((EPW,), jnp.int32),
        pltpu.VMEM((NCHUNK, CD), jnp.int32),
        pltpu.VMEM((2 * CD, D), jnp.float32),
        pltpu.VMEM((2, CD), jnp.float32),
        pltpu.VMEM((NW * 16,), jnp.float32),
        pltpu.VMEM_SHARED((NPAD, D), jnp.float32),
        pltpu.VMEM_SHARED((NPAD,), jnp.float32),
        pltpu.SemaphoreType.DMA,
        pltpu.SemaphoreType.DMA,
    ],
)
def _sc_wagg(src_hbm, dst_hbm, v_hbm, alpha_hbm, maxes_hbm, z2_hbm, z1_hbm,
             wout_hbm, dout_hbm,
             sidx_v, didx2_v, rows_v, av2_v, maxv_v, wsp, dsp, sem_a, sem_b):
    c = lax.axis_index("c")
    s = lax.axis_index("s")
    wid = c * NS + s
    pltpu.sync_copy(z2_hbm.at[pl.ds(s * RPT, RPT)], wsp.at[pl.ds(s * RPT, RPT)])
    pltpu.sync_copy(z1_hbm.at[pl.ds(s * RPT, RPT)], dsp.at[pl.ds(s * RPT, RPT)])
    pltpu.sync_copy(maxes_hbm, maxv_v)
    m = jnp.full((16,), NEG_BIG, jnp.float32)
    for w in range(NW):
        m = jnp.maximum(m, maxv_v[pl.ds(w * 16, 16)])
    gmaxv = jnp.full((16,), jnp.max(m), jnp.float32)
    pltpu.sync_copy(src_hbm.at[pl.ds(wid * EPW, EPW)], sidx_v)
    pltpu.sync_copy(dst_hbm.at[wid], didx2_v)
    plsc.subcore_barrier()

    def fire(i, base, bsel, sem):
        pltpu.async_copy(v_hbm.at[sidx_v.at[pl.ds(i * CD, CD)]],
                         rows_v.at[pl.ds(base, CD)], sem)
        pltpu.async_copy(alpha_hbm.at[pl.ds(wid * EPW + i * CD, CD)], av2_v.at[bsel], sem)

    def drain(base, bsel, sem):
        pltpu.make_async_copy(v_hbm.at[sidx_v.at[pl.ds(0, CD)]],
                              rows_v.at[pl.ds(base, CD)], sem).wait()
        pltpu.make_async_copy(alpha_hbm.at[pl.ds(0, CD)], av2_v.at[bsel],
                              sem).wait()

    def process(i, base, bsel):
        def grp(g, carry):
            a16 = jnp.exp(av2_v[bsel, pl.ds(g * 16, 16)] - gmaxv)
            av2_v[bsel, pl.ds(g * 16, 16)] = a16
            for el in range(16):
                e = base + g * 16 + el
                sc = jnp.full((16,), a16[el], jnp.float32)
                for j in range(D // 16):
                    rows_v[e, pl.ds(j * 16, 16)] = rows_v[e, pl.ds(j * 16, 16)] * sc
            return carry

        lax.fori_loop(0, CD // 16, grp, 0)
        pltpu.sync_copy(rows_v.at[pl.ds(base, CD)], wsp.at[didx2_v.at[i]], add=True)
        pltpu.sync_copy(av2_v.at[bsel], dsp.at[didx2_v.at[i]], add=True)

    fire(0, 0, 0, sem_a)

    def pair(p, carry):
        fire(2 * p + 1, CD, 1, sem_b)
        drain(0, 0, sem_a)
        process(2 * p, 0, 0)
        fire(2 * p + 2, 0, 0, sem_a)
        drain(CD, 1, sem_b)
        process(2 * p + 1, CD, 1)
        return carry

    lax.fori_loop(0, (NCHUNK - 1) // 2, pair, 0)
    drain(0, 0, sem_a)
    process(NCHUNK - 1, 0, 0)
    plsc.subcore_barrier()
    pltpu.sync_copy(wsp.at[pl.ds(s * RPT, RPT)], wout_hbm.at[c, pl.ds(s * RPT, RPT)])
    pltpu.sync_copy(dsp.at[pl.ds(s * RPT, RPT)], dout_hbm.at[c, pl.ds(s * RPT, RPT)])


BLK = 1000
GRID = N // BLK


def _tc_mm_body(x_ref, w_ref, o_ref):
    o_ref[...] = jnp.dot(x_ref[...], w_ref[...], preferred_element_type=jnp.float32)


def _tc_matmul(x, w):
    return pl.pallas_call(
        _tc_mm_body,
        grid=(GRID,),
        in_specs=[
            pl.BlockSpec((BLK, D), lambda i: (i, 0)),
            pl.BlockSpec((D, D), lambda i: (0, 0)),
        ],
        out_specs=pl.BlockSpec((BLK, D), lambda i: (i, 0)),
        out_shape=jax.ShapeDtypeStruct((N, D), jnp.float32),
    )(x, w)


def _tc_scale_body(h0_ref, degp_ref, hs_ref):
    deg = degp_ref[0] + degp_ref[1] + 1.0
    dinv = lax.rsqrt(deg)
    hs_ref[...] = h0_ref[...] * dinv


def _tc_scale(h0, degp):
    return pl.pallas_call(
        _tc_scale_body,
        grid=(GRID,),
        in_specs=[
            pl.BlockSpec((BLK, D), lambda i: (i, 0)),
            pl.BlockSpec((NC, BLK, 1), lambda i: (0, i, 0)),
        ],
        out_specs=pl.BlockSpec((BLK, D), lambda i: (i, 0)),
        out_shape=jax.ShapeDtypeStruct((N, D), jnp.float32),
    )(h0, degp)


def _tc_qkvs_body(aggp_ref, h0_ref, degp_ref, gb_ref,
                  wq_ref, bq_ref, wk_ref, bk_ref, wv_ref, bv_ref, ws_ref, bs_ref,
                  q_ref, k_ref, v_ref, s_ref):
    deg = degp_ref[0] + degp_ref[1] + 1.0
    dinv = lax.rsqrt(deg)
    agg = aggp_ref[0] + aggp_ref[1]
    h = dinv * agg + (dinv * dinv) * h0_ref[...] + gb_ref[...]
    h = jnp.where(h >= 0, h, 0.01 * h)
    q_ref[...] = jnp.dot(h, wq_ref[...], preferred_element_type=jnp.float32) + bq_ref[...]
    k_ref[...] = jnp.dot(h, wk_ref[...], preferred_element_type=jnp.float32) + bk_ref[...]
    v_ref[...] = jnp.dot(h, wv_ref[...], preferred_element_type=jnp.float32) + bv_ref[...]
    s_ref[...] = jnp.dot(h, ws_ref[...], preferred_element_type=jnp.float32) + bs_ref[...]


def _tc_qkvs(aggp, h0, degp, gb, wq, bq, wk, bk, wv, bv, ws, bs):
    wspec = pl.BlockSpec((D, D), lambda i: (0, 0))
    bspec = pl.BlockSpec((1, D), lambda i: (0, 0))
    nspec = pl.BlockSpec((BLK, D), lambda i: (i, 0))
    osh = jax.ShapeDtypeStruct((N, D), jnp.float32)
    return pl.pallas_call(
        _tc_qkvs_body,
        grid=(GRID,),
        in_specs=[
            pl.BlockSpec((NC, BLK, D), lambda i: (0, i, 0)),
            nspec,
            pl.BlockSpec((NC, BLK, 1), lambda i: (0, i, 0)),
            bspec, wspec, bspec, wspec, bspec, wspec, bspec, wspec, bspec,
        ],
        out_specs=[nspec, nspec, nspec, nspec],
        out_shape=[osh, osh, osh, osh],
    )(aggp, h0, degp, gb, wq, bq, wk, bk, wv, bv, ws, bs)


def _tc_final_body(waggp_ref, denomp_ref, s_ref, o_ref):
    denom = denomp_ref[0] + denomp_ref[1] + 1e-16
    wagg = waggp_ref[0] + waggp_ref[1]
    o_ref[...] = wagg / denom + s_ref[...]


def _tc_final(waggp, denomp, sarr):
    return pl.pallas_call(
        _tc_final_body,
        grid=(GRID,),
        in_specs=[
            pl.BlockSpec((NC, BLK, D), lambda i: (0, i, 0)),
            pl.BlockSpec((NC, BLK, 1), lambda i: (0, i, 0)),
            pl.BlockSpec((BLK, D), lambda i: (i, 0)),
        ],
        out_specs=pl.BlockSpec((BLK, D), lambda i: (i, 0)),
        out_shape=jax.ShapeDtypeStruct((N, D), jnp.float32),
    )(waggp, denomp, sarr)


def kernel(x, edge_index, gcn_W, gcn_b, Wq, bq, Wk, bk, Wv, bv, Ws, bs):
    src1 = edge_index[0]
    dst1 = edge_index[1]
    dst3 = edge_index[1].reshape(NW, NCHUNK, CD)
    z1 = jnp.zeros((NPAD,), jnp.float32)
    z2 = jnp.zeros((NPAD, D), jnp.float32)

    degp = _sc_degree(dst3, z1)
    h0 = _tc_matmul(x, gcn_W)
    degp_n = degp[:, :N].reshape(NC, N, 1)
    hs = _tc_scale(h0, degp_n)

    aggp = _sc_rowagg(src1, dst3, hs, z2)
    aggp_n = aggp[:, :N].reshape(NC, N, D)
    gb = gcn_b.reshape(1, D)
    q, k, v, sarr = _tc_qkvs(aggp_n, h0, degp_n, gb,
                             Wq, bq.reshape(1, D), Wk, bk.reshape(1, D),
                             Wv, bv.reshape(1, D), Ws, bs.reshape(1, D))

    alpha, maxes = _sc_alpha(src1, dst1, q, k)
    waggp, denomp = _sc_wagg(src1, dst3, v, alpha, maxes, z2, z1)
    out = _tc_final(waggp[:, :N].reshape(NC, N, D),
                    denomp[:, :N].reshape(NC, N, 1), sarr)
    return out


# speedup vs baseline: 20.2553x; 1.1317x over previous
import functools
import math

import jax
import jax.numpy as jnp
from jax import lax
from jax.experimental import pallas as pl
from jax.experimental.pallas import tpu as pltpu
from jax.experimental.pallas import tpu_sc as plsc

N = 10000
D = 128
E = 320000

NC = 2
NS = 16
NW = NC * NS
EPW = E // NW
CD = 80
NCHUNK = EPW // CD
NPAD = 10240
RPT = NPAD // NS

INV_SQRT_D = 1.0 / math.sqrt(D)
NEG_BIG = -3.0e38

_MESH = plsc.VectorSubcoreMesh(core_axis_name="c", subcore_axis_name="s")
_PARAMS = pltpu.CompilerParams(needs_layout_passes=False)


@functools.partial(
    pl.kernel,
    compiler_params=_PARAMS,
    out_type=jax.ShapeDtypeStruct((NC, NPAD), jnp.float32),
    mesh=_MESH,
    scratch_types=[
        pltpu.VMEM((NCHUNK, CD), jnp.int32),
        pltpu.VMEM((CD,), jnp.float32),
        pltpu.VMEM_SHARED((NPAD,), jnp.float32),
    ],
)
def _sc_degree(dst_hbm, z1_hbm, out_hbm, didx2_v, ones_v, deg_sp):
    c = lax.axis_index("c")
    s = lax.axis_index("s")
    wid = c * NS + s
    pltpu.sync_copy(z1_hbm.at[pl.ds(s * RPT, RPT)], deg_sp.at[pl.ds(s * RPT, RPT)])
    pltpu.sync_copy(dst_hbm.at[wid], didx2_v)
    for g in range(CD // 16):
        ones_v[pl.ds(g * 16, 16)] = jnp.full((16,), 1.0, jnp.float32)
    plsc.subcore_barrier()

    def chunk(i, carry):
        pltpu.sync_copy(ones_v, deg_sp.at[didx2_v.at[i]], add=True)
        return carry

    lax.fori_loop(0, NCHUNK, chunk, 0)
    plsc.subcore_barrier()
    pltpu.sync_copy(deg_sp.at[pl.ds(s * RPT, RPT)], out_hbm.at[c, pl.ds(s * RPT, RPT)])


@functools.partial(
    pl.kernel,
    compiler_params=_PARAMS,
    out_type=jax.ShapeDtypeStruct((NC, NPAD, D), jnp.float32),
    mesh=_MESH,
    scratch_types=[
        pltpu.VMEM((EPW,), jnp.int32),
        pltpu.VMEM((NCHUNK, CD), jnp.int32),
        pltpu.VMEM((2 * CD, D), jnp.float32),
        pltpu.VMEM_SHARED((NPAD, D), jnp.float32),
        pltpu.SemaphoreType.DMA,
        pltpu.SemaphoreType.DMA,
    ],
)
def _sc_rowagg(src_hbm, dst_hbm, hs_hbm, z2_hbm, out_hbm,
               sidx_v, didx2_v, rows_v, agg_sp, sem_a, sem_b):
    c = lax.axis_index("c")
    s = lax.axis_index("s")
    wid = c * NS + s
    pltpu.sync_copy(z2_hbm.at[pl.ds(s * RPT, RPT)], agg_sp.at[pl.ds(s * RPT, RPT)])
    pltpu.sync_copy(src_hbm.at[pl.ds(wid * EPW, EPW)], sidx_v)
    pltpu.sync_copy(dst_hbm.at[wid], didx2_v)
    plsc.subcore_barrier()

    def fire(i, base, sem):
        pltpu.async_copy(hs_hbm.at[sidx_v.at[pl.ds(i * CD, CD)]],
                         rows_v.at[pl.ds(base, CD)], sem)

    def drain(base, sem):
        pltpu.make_async_copy(hs_hbm.at[sidx_v.at[pl.ds(0, CD)]],
                              rows_v.at[pl.ds(base, CD)], sem).wait()

    def flush(i, base):
        pltpu.sync_copy(rows_v.at[pl.ds(base, CD)], agg_sp.at[didx2_v.at[i]], add=True)

    fire(0, 0, sem_a)

    def pair(p, carry):
        fire(2 * p + 1, CD, sem_b)
        drain(0, sem_a)
        flush(2 * p, 0)
        fire(2 * p + 2, 0, sem_a)
        drain(CD, sem_b)
        flush(2 * p + 1, CD)
        return carry

    lax.fori_loop(0, (NCHUNK - 1) // 2, pair, 0)
    drain(0, sem_a)
    flush(NCHUNK - 1, 0)
    plsc.subcore_barrier()
    pltpu.sync_copy(agg_sp.at[pl.ds(s * RPT, RPT)], out_hbm.at[c, pl.ds(s * RPT, RPT)])


@functools.partial(
    pl.kernel,
    compiler_params=_PARAMS,
    out_type=(
        jax.ShapeDtypeStruct((E,), jnp.float32),
        jax.ShapeDtypeStruct((NW * 16,), jnp.float32),
    ),
    mesh=_MESH,
    scratch_types=[
        pltpu.VMEM((EPW,), jnp.int32),
        pltpu.VMEM((EPW,), jnp.int32),
        pltpu.VMEM((2 * CD, D), jnp.float32),
        pltpu.VMEM((2 * CD, D), jnp.float32),
        pltpu.VMEM((EPW,), jnp.float32),
        pltpu.VMEM((16,), jnp.float32),
        pltpu.SemaphoreType.DMA,
        pltpu.SemaphoreType.DMA,
    ],
)
def _sc_alpha(src_hbm, dst_hbm, q_hbm, k_hbm, alpha_hbm, maxes_hbm,
              sidx_v, didx_v, q_v, k_v, av_v, mv_v, sem_a, sem_b):
    c = lax.axis_index("c")
    s = lax.axis_index("s")
    wid = c * NS + s
    pltpu.sync_copy(src_hbm.at[pl.ds(wid * EPW, EPW)], sidx_v)
    pltpu.sync_copy(dst_hbm.at[pl.ds(wid * EPW, EPW)], didx_v)
    lanes = lax.iota(jnp.int32, 16)

    def fire(i, base, sem):
        pltpu.async_copy(q_hbm.at[didx_v.at[pl.ds(i * CD, CD)]],
                         q_v.at[pl.ds(base, CD)], sem)
        pltpu.async_copy(k_hbm.at[sidx_v.at[pl.ds(i * CD, CD)]],
                         k_v.at[pl.ds(base, CD)], sem)

    def drain(base, sem):
        pltpu.make_async_copy(q_hbm.at[didx_v.at[pl.ds(0, CD)]],
                              q_v.at[pl.ds(base, CD)], sem).wait()
        pltpu.make_async_copy(k_hbm.at[sidx_v.at[pl.ds(0, CD)]],
                              k_v.at[pl.ds(base, CD)], sem).wait()

    perms = [jnp.bitwise_xor(lanes, sh) for sh in (8, 4, 2, 1)]

    def compute(i, base, m):
        def grp(g, m2):
            parts = []
            for el in range(16):
                e = base + g * 16 + el
                acc = q_v[e, pl.ds(0, 16)] * k_v[e, pl.ds(0, 16)]
                for j in range(1, D // 16):
                    acc = acc + q_v[e, pl.ds(j * 16, 16)] * k_v[e, pl.ds(j * 16, 16)]
                for perm in perms:
                    acc = acc + acc[perm]
                parts.append(jnp.where(lanes == el, acc, 0.0))
            while len(parts) > 1:
                parts = [parts[2 * t] + parts[2 * t + 1]
                         for t in range(len(parts) // 2)]
            av = parts[0] * INV_SQRT_D
            av_v[pl.ds(i * CD + g * 16, 16)] = av
            return jnp.maximum(m2, av)

        return lax.fori_loop(0, CD // 16, grp, m)

    fire(0, 0, sem_a)

    def pair(p, m):
        fire(2 * p + 1, CD, sem_b)
        drain(0, sem_a)
        m = compute(2 * p, 0, m)
        fire(2 * p + 2, 0, sem_a)
        drain(CD, sem_b)
        m = compute(2 * p + 1, CD, m)
        return m

    m = lax.fori_loop(0, (NCHUNK - 1) // 2, pair,
                      jnp.full((16,), NEG_BIG, jnp.float32))
    drain(0, sem_a)
    m = compute(NCHUNK - 1, 0, m)
    pltpu.sync_copy(av_v, alpha_hbm.at[pl.ds(wid * EPW, EPW)])
    mv_v[...] = m
    pltpu.sync_copy(mv_v, maxes_hbm.at[pl.ds(wid * 16, 16)])


@functools.partial(
    pl.kernel,
    compiler_params=_PARAMS,
    out_type=(
        jax.ShapeDtypeStruct((NC, NPAD, D), jnp.float32),
        jax.ShapeDtypeStruct((NC, NPAD), jnp.float32),
    ),
    mesh=_MESH,
    scratch_types=[
        pltpu.VMEM((EPW,), jnp.int32),
        pltpu.VMEM((NCHUNK, CD), jnp.int32),
        pltpu.VMEM((2 * CD, D), jnp.float32),
        pltpu.VMEM((2, CD), jnp.float32),
        pltpu.VMEM((NW * 16,), jnp.float32),
        pltpu.VMEM_SHARED((NPAD, D), jnp.float32),
        pltpu.VMEM_SHARED((NPAD,), jnp.float32),
        pltpu.SemaphoreType.DMA,
        pltpu.SemaphoreType.DMA,
    ],
)
def _sc_wagg(src_hbm, dst_hbm, v_hbm, alpha_hbm, maxes_hbm, z2_hbm, z1_hbm,
             wout_hbm, dout_hbm,
             sidx_v, didx2_v, rows_v, av2_v, maxv_v, wsp, dsp, sem_a, sem_b):
    c = lax.axis_index("c")
    s = lax.axis_index("s")
    wid = c * NS + s
    pltpu.sync_copy(z2_hbm.at[pl.ds(s * RPT, RPT)], wsp.at[pl.ds(s * RPT, RPT)])
    pltpu.sync_copy(z1_hbm.at[pl.ds(s * RPT, RPT)], dsp.at[pl.ds(s * RPT, RPT)])
    pltpu.sync_copy(maxes_hbm, maxv_v)
    m = jnp.full((16,), NEG_BIG, jnp.float32)
    for w in range(NW):
        m = jnp.maximum(m, maxv_v[pl.ds(w * 16, 16)])
    gmaxv = jnp.full((16,), jnp.max(m), jnp.float32)
    pltpu.sync_copy(src_hbm.at[pl.ds(wid * EPW, EPW)], sidx_v)
    pltpu.sync_copy(dst_hbm.at[wid], didx2_v)
    plsc.subcore_barrier()

    def fire(i, base, bsel, sem):
        pltpu.async_copy(v_hbm.at[sidx_v.at[pl.ds(i * CD, CD)]],
                         rows_v.at[pl.ds(base, CD)], sem)
        pltpu.async_copy(alpha_hbm.at[pl.ds(wid * EPW + i * CD, CD)], av2_v.at[bsel], sem)

    def drain(base, bsel, sem):
        pltpu.make_async_copy(v_hbm.at[sidx_v.at[pl.ds(0, CD)]],
                              rows_v.at[pl.ds(base, CD)], sem).wait()
        pltpu.make_async_copy(alpha_hbm.at[pl.ds(0, CD)], av2_v.at[bsel],
                              sem).wait()

    def process(i, base, bsel):
        def grp(g, carry):
            a16 = jnp.exp(av2_v[bsel, pl.ds(g * 16, 16)] - gmaxv)
            av2_v[bsel, pl.ds(g * 16, 16)] = a16
            for el in range(16):
                e = base + g * 16 + el
                sc = jnp.full((16,), a16[el], jnp.float32)
                for j in range(D // 16):
                    rows_v[e, pl.ds(j * 16, 16)] = rows_v[e, pl.ds(j * 16, 16)] * sc
            return carry

        lax.fori_loop(0, CD // 16, grp, 0)
        pltpu.sync_copy(rows_v.at[pl.ds(base, CD)], wsp.at[didx2_v.at[i]], add=True)
        pltpu.sync_copy(av2_v.at[bsel], dsp.at[didx2_v.at[i]], add=True)

    fire(0, 0, 0, sem_a)

    def pair(p, carry):
        fire(2 * p + 1, CD, 1, sem_b)
        drain(0, 0, sem_a)
        process(2 * p, 0, 0)
        fire(2 * p + 2, 0, 0, sem_a)
        drain(CD, 1, sem_b)
        process(2 * p + 1, CD, 1)
        return carry

    lax.fori_loop(0, (NCHUNK - 1) // 2, pair, 0)
    drain(0, 0, sem_a)
    process(NCHUNK - 1, 0, 0)
    plsc.subcore_barrier()
    pltpu.sync_copy(wsp.at[pl.ds(s * RPT, RPT)], wout_hbm.at[c, pl.ds(s * RPT, RPT)])
    pltpu.sync_copy(dsp.at[pl.ds(s * RPT, RPT)], dout_hbm.at[c, pl.ds(s * RPT, RPT)])


BLK = 1000
GRID = N // BLK


def _tc_mm_body(x_ref, w_ref, o_ref):
    o_ref[...] = jnp.dot(x_ref[...], w_ref[...], preferred_element_type=jnp.float32)


def _tc_matmul(x, w):
    return pl.pallas_call(
        _tc_mm_body,
        grid=(GRID,),
        in_specs=[
            pl.BlockSpec((BLK, D), lambda i: (i, 0)),
            pl.BlockSpec((D, D), lambda i: (0, 0)),
        ],
        out_specs=pl.BlockSpec((BLK, D), lambda i: (i, 0)),
        out_shape=jax.ShapeDtypeStruct((N, D), jnp.float32),
    )(x, w)


def _tc_scale_body(h0_ref, degp_ref, hs_ref):
    deg = degp_ref[0] + degp_ref[1] + 1.0
    dinv = lax.rsqrt(deg)
    hs_ref[...] = h0_ref[...] * dinv


def _tc_scale(h0, degp):
    return pl.pallas_call(
        _tc_scale_body,
        grid=(GRID,),
        in_specs=[
            pl.BlockSpec((BLK, D), lambda i: (i, 0)),
            pl.BlockSpec((NC, BLK, 1), lambda i: (0, i, 0)),
        ],
        out_specs=pl.BlockSpec((BLK, D), lambda i: (i, 0)),
        out_shape=jax.ShapeDtypeStruct((N, D), jnp.float32),
    )(h0, degp)


def _tc_qkvs_body(aggp_ref, h0_ref, degp_ref, gb_ref,
                  wq_ref, bq_ref, wk_ref, bk_ref, wv_ref, bv_ref, ws_ref, bs_ref,
                  q_ref, k_ref, v_ref, s_ref):
    deg = degp_ref[0] + degp_ref[1] + 1.0
    dinv = lax.rsqrt(deg)
    agg = aggp_ref[0] + aggp_ref[1]
    h = dinv * agg + (dinv * dinv) * h0_ref[...] + gb_ref[...]
    h = jnp.where(h >= 0, h, 0.01 * h)
    q_ref[...] = jnp.dot(h, wq_ref[...], preferred_element_type=jnp.float32) + bq_ref[...]
    k_ref[...] = jnp.dot(h, wk_ref[...], preferred_element_type=jnp.float32) + bk_ref[...]
    v_ref[...] = jnp.dot(h, wv_ref[...], preferred_element_type=jnp.float32) + bv_ref[...]
    s_ref[...] = jnp.dot(h, ws_ref[...], preferred_element_type=jnp.float32) + bs_ref[...]


def _tc_qkvs(aggp, h0, degp, gb, wq, bq, wk, bk, wv, bv, ws, bs):
    wspec = pl.BlockSpec((D, D), lambda i: (0, 0))
    bspec = pl.BlockSpec((1, D), lambda i: (0, 0))
    nspec = pl.BlockSpec((BLK, D), lambda i: (i, 0))
    osh = jax.ShapeDtypeStruct((N, D), jnp.float32)
    return pl.pallas_call(
        _tc_qkvs_body,
        grid=(GRID,),
        in_specs=[
            pl.BlockSpec((NC, BLK, D), lambda i: (0, i, 0)),
            nspec,
            pl.BlockSpec((NC, BLK, 1), lambda i: (0, i, 0)),
            bspec, wspec, bspec, wspec, bspec, wspec, bspec, wspec, bspec,
        ],
        out_specs=[nspec, nspec, nspec, nspec],
        out_shape=[osh, osh, osh, osh],
    )(aggp, h0, degp, gb, wq, bq, wk, bk, wv, bv, ws, bs)


def _tc_final_body(waggp_ref, denomp_ref, s_ref, o_ref):
    denom = denomp_ref[0] + denomp_ref[1] + 1e-16
    wagg = waggp_ref[0] + waggp_ref[1]
    o_ref[...] = wagg / denom + s_ref[...]


def _tc_final(waggp, denomp, sarr):
    return pl.pallas_call(
        _tc_final_body,
        grid=(GRID,),
        in_specs=[
            pl.BlockSpec((NC, BLK, D), lambda i: (0, i, 0)),
            pl.BlockSpec((NC, BLK, 1), lambda i: (0, i, 0)),
            pl.BlockSpec((BLK, D), lambda i: (i, 0)),
        ],
        out_specs=pl.BlockSpec((BLK, D), lambda i: (i, 0)),
        out_shape=jax.ShapeDtypeStruct((N, D), jnp.float32),
    )(waggp, denomp, sarr)


def kernel(x, edge_index, gcn_W, gcn_b, Wq, bq, Wk, bk, Wv, bv, Ws, bs):
    src1 = edge_index[0]
    dst1 = edge_index[1]
    dst3 = edge_index[1].reshape(NW, NCHUNK, CD)
    z1 = jnp.zeros((NPAD,), jnp.float32)
    z2 = jnp.zeros((NPAD, D), jnp.float32)

    degp = _sc_degree(dst3, z1)
    h0 = _tc_matmul(x, gcn_W)
    degp_n = degp.reshape(NC, NPAD, 1)
    hs = _tc_scale(h0, degp_n)

    aggp = _sc_rowagg(src1, dst3, hs, z2)
    gb = gcn_b.reshape(1, D)
    q, k, v, sarr = _tc_qkvs(aggp, h0, degp_n, gb,
                             Wq, bq.reshape(1, D), Wk, bk.reshape(1, D),
                             Wv, bv.reshape(1, D), Ws, bs.reshape(1, D))

    alpha, maxes = _sc_alpha(src1, dst1, q, k)
    waggp, denomp = _sc_wagg(src1, dst3, v, alpha, maxes, z2, z1)
    out = _tc_final(waggp, denomp.reshape(NC, NPAD, 1), sarr)
    return out
